# Initial kernel scaffold; baseline (speedup 1.0000x reference)
import jax
import jax.numpy as jnp
from jax.experimental import pallas as pl


def kernel(seq, edge_index, edge_vals, W, b):
    raise NotImplementedError("write your pallas kernel here")


# speedup vs baseline: 2.1941x; 2.1941x over previous
import functools

import jax
import jax.numpy as jnp
from jax import lax
from jax.experimental import pallas as pl
from jax.experimental.pallas import tpu as pltpu
from jax.experimental.pallas import tpu_sc as plsc

D = 128
HF = 64
LANES = 16
CHUNK = 512
SUB = CHUNK // 128
Z_ROWS = 128


def _sc_scatter(seq_lo, seq_hi, src2d, dst2d, vals, n_pad,
                num_cores, num_subcores):
    nw = num_cores * num_subcores
    e_pad = vals.shape[0]
    epw = e_pad // nw
    n_chunks = epw // CHUNK
    rows_per_tile = n_pad // num_subcores
    z_copies = rows_per_tile // Z_ROWS

    mesh = plsc.VectorSubcoreMesh(core_axis_name="c", subcore_axis_name="s")

    @functools.partial(
        pl.kernel,
        out_type=jax.ShapeDtypeStruct((num_cores, 2, n_pad, HF),
                                      jnp.float32),
        mesh=mesh,
        scratch_types=[
            pltpu.VMEM((SUB, 128), jnp.int32),
            pltpu.VMEM((SUB, 128), jnp.int32),
            pltpu.VMEM((CHUNK,), jnp.float32),
            pltpu.VMEM((CHUNK, HF), jnp.float32),
            pltpu.VMEM((Z_ROWS, HF), jnp.float32),
            pltpu.VMEM_SHARED((n_pad, HF), jnp.float32),
            pltpu.SemaphoreType.DMA,
        ],
        compiler_params=pltpu.CompilerParams(use_tc_tiling_on_sc=False),
    )
    def body(seq_lo_h, seq_hi_h, src_h, dst_h, val_h, out_h,
             src_v, dst_v, val_v, rows_v, zero_v, acc, sem):
        c = lax.axis_index("c")
        s = lax.axis_index("s")
        wid = s * num_cores + c
        row0 = s * rows_per_tile

        def zfill(r, carry):
            for f in range(HF // LANES):
                zero_v[r, pl.ds(f * LANES, LANES)] = jnp.zeros(
                    (LANES,), jnp.float32)
            return carry
        lax.fori_loop(0, Z_ROWS, zfill, 0)

        for half, seq_h in enumerate((seq_lo_h, seq_hi_h)):
            for k in range(z_copies):
                pltpu.sync_copy(zero_v,
                                acc.at[pl.ds(row0 + k * Z_ROWS, Z_ROWS)])
            plsc.subcore_barrier()

            def chunk_body(i, carry):
                erow0 = wid * (epw // 128) + i * SUB
                pltpu.sync_copy(src_h.at[pl.ds(erow0, SUB)], src_v)
                pltpu.sync_copy(dst_h.at[pl.ds(erow0, SUB)], dst_v)
                pltpu.sync_copy(
                    val_h.at[pl.ds(wid * epw + i * CHUNK, CHUNK)], val_v)
                copies = [
                    pltpu.async_copy(seq_h.at[src_v.at[j]],
                                     rows_v.at[pl.ds(j * 128, 128)], sem)
                    for j in range(SUB)
                ]
                for cp in copies:
                    cp.wait()

                def scale_body(g, carry2):
                    e0 = g * LANES
                    val16 = val_v[pl.ds(e0, LANES)]
                    for j in range(LANES):
                        vb = jnp.full((LANES,), val16[j], jnp.float32)
                        for f in range(HF // LANES):
                            sl = pl.ds(f * LANES, LANES)
                            rows_v[e0 + j, sl] = rows_v[e0 + j, sl] * vb
                    return carry2
                lax.fori_loop(0, CHUNK // LANES, scale_body, 0)

                for j in range(SUB):
                    pltpu.sync_copy(rows_v.at[pl.ds(j * 128, 128)],
                                    acc.at[dst_v.at[j]], add=True)
                return carry
            lax.fori_loop(0, n_chunks, chunk_body, 0)
            plsc.subcore_barrier()

            pltpu.sync_copy(acc.at[pl.ds(row0, rows_per_tile)],
                            out_h.at[c, half, pl.ds(row0, rows_per_tile)])

    return body(seq_lo, seq_hi, src2d, dst2d, vals)


def _tc_finish(partials, W, b2d, n):
    nc = partials.shape[0]
    br = 1000
    grid = n // br

    def tc_body(p_ref, w_ref, b_ref, o_ref):
        xs = []
        for half in range(2):
            xh = p_ref[0, half]
            for k in range(1, nc):
                xh = xh + p_ref[k, half]
            xs.append(xh)
        x = jnp.concatenate(xs, axis=1)
        y = lax.dot_general(x, w_ref[...], (((1,), (1,)), ((), ())),
                            preferred_element_type=jnp.float32)
        o_ref[...] = jnp.maximum(y + b_ref[...], 0.0)

    return pl.pallas_call(
        tc_body,
        grid=(grid,),
        in_specs=[
            pl.BlockSpec((nc, 2, br, HF), lambda i: (0, 0, i, 0)),
            pl.BlockSpec((D, D), lambda i: (0, 0)),
            pl.BlockSpec((1, D), lambda i: (0, 0)),
        ],
        out_specs=pl.BlockSpec((br, D), lambda i: (i, 0)),
        out_shape=jax.ShapeDtypeStruct((n, D), jnp.float32),
    )(partials, W, b2d)


def kernel(seq, edge_index, edge_vals, W, b):
    n, _ = seq.shape
    info = plsc.get_sparse_core_info()
    nc, ns = info.num_cores, info.num_subcores
    nw = nc * ns
    e = edge_vals.shape[0]
    epw = -(-e // nw)
    epw = -(-epw // CHUNK) * CHUNK
    e_pad = epw * nw
    pad = e_pad - e
    n_pad = -(-n // (ns * Z_ROWS)) * (ns * Z_ROWS)

    dst = edge_index[0].astype(jnp.int32)
    src = edge_index[1].astype(jnp.int32)
    src2d = jnp.pad(src, (0, pad)).reshape(-1, 128)
    dst2d = jnp.pad(dst, (0, pad)).reshape(-1, 128)
    val_p = jnp.pad(edge_vals, (0, pad))

    partials = _sc_scatter(seq[:, :HF], seq[:, HF:], src2d, dst2d, val_p,
                           n_pad, nc, ns)
    return _tc_finish(partials, W, b.reshape(1, D), n)


# speedup vs baseline: 3.4474x; 1.5712x over previous
import functools

import jax
import jax.numpy as jnp
from jax import lax
from jax.experimental import pallas as pl
from jax.experimental.pallas import tpu as pltpu
from jax.experimental.pallas import tpu_sc as plsc

D = 128
HF = 64
LANES = 16
CHUNK = 256
SUB = CHUNK // 128
Z_ROWS = 128


def _sc_scatter(seq_lo, seq_hi, src2d, dst2d, vals, n_pad,
                num_cores, num_subcores):
    nw = num_cores * num_subcores
    e_pad = vals.shape[0]
    epw = e_pad // nw
    n_chunks = epw // CHUNK
    rows_per_tile = n_pad // num_subcores
    z_copies = rows_per_tile // Z_ROWS

    mesh = plsc.VectorSubcoreMesh(core_axis_name="c", subcore_axis_name="s")

    @functools.partial(
        pl.kernel,
        out_type=jax.ShapeDtypeStruct((num_cores, 2, n_pad, HF),
                                      jnp.float32),
        mesh=mesh,
        scratch_types=[
            pltpu.VMEM((epw // 128, 128), jnp.int32),
            pltpu.VMEM((epw // 128, 128), jnp.int32),
            pltpu.VMEM((epw,), jnp.float32),
            pltpu.VMEM((CHUNK, HF), jnp.float32),
            pltpu.VMEM((CHUNK, HF), jnp.float32),
            pltpu.VMEM((Z_ROWS, HF), jnp.float32),
            pltpu.VMEM_SHARED((n_pad, HF), jnp.float32),
            pltpu.SemaphoreType.DMA,
            pltpu.SemaphoreType.DMA,
            pltpu.SemaphoreType.DMA,
            pltpu.SemaphoreType.DMA,
        ],
        compiler_params=pltpu.CompilerParams(use_tc_tiling_on_sc=False),
    )
    def body(seq_lo_h, seq_hi_h, src_h, dst_h, val_h, out_h,
             src_v, dst_v, val_v, rows0, rows1, zero_v, acc,
             sem_g0, sem_g1, sem_s0, sem_s1):
        c = lax.axis_index("c")
        s = lax.axis_index("s")
        wid = s * num_cores + c
        row0 = s * rows_per_tile
        rpw = epw // 128

        pltpu.sync_copy(src_h.at[pl.ds(wid * rpw, rpw)], src_v)
        pltpu.sync_copy(dst_h.at[pl.ds(wid * rpw, rpw)], dst_v)
        pltpu.sync_copy(val_h.at[pl.ds(wid * epw, epw)], val_v)

        def zfill(r, carry):
            for f in range(HF // LANES):
                zero_v[r, pl.ds(f * LANES, LANES)] = jnp.zeros(
                    (LANES,), jnp.float32)
            return carry
        lax.fori_loop(0, Z_ROWS, zfill, 0)

        def scale(rows_v, ebase):
            def scale_body(g, carry2):
                e0 = g * LANES
                val16 = val_v[pl.ds(ebase + e0, LANES)]
                for j in range(LANES):
                    vb = jnp.full((LANES,), val16[j], jnp.float32)
                    for f in range(HF // LANES):
                        sl = pl.ds(f * LANES, LANES)
                        rows_v[e0 + j, sl] = rows_v[e0 + j, sl] * vb
                return carry2
            lax.fori_loop(0, CHUNK // LANES, scale_body, 0, unroll=2)

        for half, seq_h in enumerate((seq_lo_h, seq_hi_h)):
            for k in range(z_copies):
                pltpu.sync_copy(zero_v,
                                acc.at[pl.ds(row0 + k * Z_ROWS, Z_ROWS)])
            plsc.subcore_barrier()

            def pair_body(p, carry):
                k0 = 2 * p
                g0 = [
                    pltpu.async_copy(seq_h.at[src_v.at[k0 * SUB + j]],
                                     rows0.at[pl.ds(j * 128, 128)], sem_g0)
                    for j in range(SUB)
                ]
                g1 = [
                    pltpu.async_copy(seq_h.at[src_v.at[(k0 + 1) * SUB + j]],
                                     rows1.at[pl.ds(j * 128, 128)], sem_g1)
                    for j in range(SUB)
                ]
                for cp in g0:
                    cp.wait()
                scale(rows0, k0 * CHUNK)
                s0 = [
                    pltpu.async_copy(rows0.at[pl.ds(j * 128, 128)],
                                     acc.at[dst_v.at[k0 * SUB + j]],
                                     sem_s0, add=True)
                    for j in range(SUB)
                ]
                for cp in g1:
                    cp.wait()
                scale(rows1, (k0 + 1) * CHUNK)
                s1 = [
                    pltpu.async_copy(rows1.at[pl.ds(j * 128, 128)],
                                     acc.at[dst_v.at[(k0 + 1) * SUB + j]],
                                     sem_s1, add=True)
                    for j in range(SUB)
                ]
                for cp in s0:
                    cp.wait()
                for cp in s1:
                    cp.wait()
                return carry
            lax.fori_loop(0, n_chunks // 2, pair_body, 0)
            plsc.subcore_barrier()

            pltpu.sync_copy(acc.at[pl.ds(row0, rows_per_tile)],
                            out_h.at[c, half, pl.ds(row0, rows_per_tile)])

    return body(seq_lo, seq_hi, src2d, dst2d, vals)


def _tc_finish(partials, W, b2d, n):
    nc = partials.shape[0]
    br = 1000
    grid = n // br

    def tc_body(p_ref, w_ref, b_ref, o_ref):
        xs = []
        for half in range(2):
            xh = p_ref[0, half]
            for k in range(1, nc):
                xh = xh + p_ref[k, half]
            xs.append(xh)
        x = jnp.concatenate(xs, axis=1)
        y = lax.dot_general(x, w_ref[...], (((1,), (1,)), ((), ())),
                            preferred_element_type=jnp.float32)
        o_ref[...] = jnp.maximum(y + b_ref[...], 0.0)

    return pl.pallas_call(
        tc_body,
        grid=(grid,),
        in_specs=[
            pl.BlockSpec((nc, 2, br, HF), lambda i: (0, 0, i, 0)),
            pl.BlockSpec((D, D), lambda i: (0, 0)),
            pl.BlockSpec((1, D), lambda i: (0, 0)),
        ],
        out_specs=pl.BlockSpec((br, D), lambda i: (i, 0)),
        out_shape=jax.ShapeDtypeStruct((n, D), jnp.float32),
    )(partials, W, b2d)


def kernel(seq, edge_index, edge_vals, W, b):
    n, _ = seq.shape
    info = plsc.get_sparse_core_info()
    nc, ns = info.num_cores, info.num_subcores
    nw = nc * ns
    e = edge_vals.shape[0]
    epw = -(-e // nw)
    epw = -(-epw // CHUNK) * CHUNK
    e_pad = epw * nw
    pad = e_pad - e
    n_pad = -(-n // (ns * Z_ROWS)) * (ns * Z_ROWS)

    dst = edge_index[0].astype(jnp.int32)
    src = edge_index[1].astype(jnp.int32)
    src2d = jnp.pad(src, (0, pad)).reshape(-1, 128)
    dst2d = jnp.pad(dst, (0, pad)).reshape(-1, 128)
    val_p = jnp.pad(edge_vals, (0, pad))

    partials = _sc_scatter(seq[:, :HF], seq[:, HF:], src2d, dst2d, val_p,
                           n_pad, nc, ns)
    return _tc_finish(partials, W, b.reshape(1, D), n)
